# Initial kernel scaffold; baseline (speedup 1.0000x reference)
import jax
import jax.numpy as jnp
from jax.experimental import pallas as pl


def kernel(q_rot, spike, time_norm, var_id, mask, params):
    raise NotImplementedError("write your pallas kernel here")


# speedup vs baseline: 21.9456x; 21.9456x over previous
import functools
import math

import jax
import jax.numpy as jnp
from jax.experimental import pallas as pl
from jax.experimental.pallas import tpu as pltpu

D = 256
KE = 128
NH = 4
HD = D // NH
TN = 512


def _build_wt(p):
    r, i, j, k = p['r'], p['i'], p['j'], p['k']
    W = jnp.concatenate([
        jnp.concatenate([r, -i, -j, -k], 1),
        jnp.concatenate([i, r, -k, j], 1),
        jnp.concatenate([j, k, r, -i], 1),
        jnp.concatenate([k, -j, i, r], 1),
    ], 0)
    return W.T, p['b'].reshape(1, -1)


def _excl_cumsum_2d(x, su, sl):
    inrow = jax.lax.dot_general(x, su, (((1,), (0,)), ((), ())),
                                preferred_element_type=jnp.float32)
    tot = jnp.sum(x, axis=1, keepdims=True)
    off = jax.lax.dot_general(sl, tot, (((1,), (0,)), ((), ())),
                              preferred_element_type=jnp.float32)
    return inrow + off


def _select_body(spike_ref, mask_ref, posq_ref):
    v = spike_ref[0] * mask_ref[0]
    b0 = jax.lax.bitcast_convert_type(v, jnp.int32)
    key = b0 ^ ((b0 >> 31) & jnp.int32(0x7FFFFFFF))

    k_target = jnp.int32(KE)

    def bs_body(_, carry):
        lo, hi = carry
        mid = (lo & hi) + ((lo ^ hi) >> 1)
        cnt = jnp.sum((key >= mid).astype(jnp.int32))
        ge = cnt >= k_target
        return (jnp.where(ge, mid, lo), jnp.where(ge, hi, mid))

    lo, _ = jax.lax.fori_loop(
        0, 32, bs_body,
        (jnp.int32(-2**31), jnp.int32(2**31 - 1)))
    tau = lo

    gt = (key > tau).astype(jnp.float32)
    eq = (key == tau).astype(jnp.float32)
    c = jnp.sum(gt)
    r_need = jnp.float32(KE) - c

    rows = spike_ref.shape[1]
    iota_r = jax.lax.broadcasted_iota(jnp.int32, (128, 128), 0)
    iota_c = jax.lax.broadcasted_iota(jnp.int32, (128, 128), 1)
    su = (iota_r < iota_c).astype(jnp.float32)
    ir = jax.lax.broadcasted_iota(jnp.int32, (rows, rows), 0)
    ic = jax.lax.broadcasted_iota(jnp.int32, (rows, rows), 1)
    sl = (ir > ic).astype(jnp.float32)

    eqpos = _excl_cumsum_2d(eq, su, sl)
    sel = gt + eq * (eqpos < r_need).astype(jnp.float32)
    pos = _excl_cumsum_2d(sel, su, sl)
    posq_ref[0] = jnp.where(sel > 0.5, pos, jnp.float32(3e7))


def _events_body(posq_ref, t_ref, var_ref, pb_ref, sb_ref, tb_ref, vb_ref,
                 mb_ref, q_ref, dta_ref, wst_ref, bs_ref, wat_ref, ba_ref,
                 wkt_ref, bk_ref, wvt_ref, bv_ref,
                 k_out, v_out, ts_out,
                 ts_scr, vs_scr, num_scr, den_scr, qseed_scr,
                 *, nblk):
    i = pl.program_id(1)
    q = q_ref[0]
    dt_a = dta_ref[0, 0]

    kiota = jax.lax.broadcasted_iota(jnp.int32, (KE, TN), 0).astype(jnp.float32)

    @pl.when(i == 0)
    def _init():
        pr = posq_ref[0]
        trow = t_ref[0]
        vrow = var_ref[0]
        ts = jnp.zeros((KE, 1), jnp.float32)
        vs = jnp.zeros((KE, 1), jnp.float32)
        for j in range(nblk):
            sj = (kiota == pr[:, j * TN:(j + 1) * TN]).astype(jnp.float32)
            ts += jnp.sum(sj * trow[:, j * TN:(j + 1) * TN], axis=1,
                          keepdims=True)
            vs += jnp.sum(sj * vrow[:, j * TN:(j + 1) * TN], axis=1,
                          keepdims=True)
        ts_scr[...] = ts
        vs_scr[...] = vs
        num_scr[...] = jnp.zeros_like(num_scr)
        den_scr[...] = jnp.zeros_like(den_scr)
        qseed_scr[...] = jnp.zeros_like(qseed_scr)

    tb = tb_ref[0]
    vb = vb_ref[0]
    sb = sb_ref[0]
    mb = mb_ref[0]
    pb = pb_ref[0]

    s_blk = (kiota == pb).astype(jnp.float32)
    qseed_scr[...] += jax.lax.dot_general(
        s_blk, q, (((1,), (0,)), ((), ())), preferred_element_type=jnp.float32)

    delta = jnp.abs(tb - ts_scr[...])
    win = (delta <= dt_a).astype(jnp.float32)
    decay = jnp.exp(-2.0 * delta / jnp.maximum(dt_a, 0.001))
    vaff = 0.5 + 0.5 * (vb == vs_scr[...]).astype(jnp.float32)
    incid = win * decay * vaff * sb * mb
    num_scr[...] += jax.lax.dot_general(
        incid, q, (((1,), (0,)), ((), ())), preferred_element_type=jnp.float32)
    den_scr[...] += jnp.sum(incid, axis=1, keepdims=True)

    @pl.when(i == nblk - 1)
    def _fin():
        hs = jnp.dot(qseed_scr[...], wst_ref[...],
                     preferred_element_type=jnp.float32) + bs_ref[...]
        agg = num_scr[...] / jnp.maximum(den_scr[...], 1e-6)
        he = hs + jnp.dot(agg, wat_ref[...],
                          preferred_element_type=jnp.float32) + ba_ref[...]
        k_out[0] = jnp.dot(he, wkt_ref[...],
                           preferred_element_type=jnp.float32) + bk_ref[...]
        v_out[0] = jnp.dot(he, wvt_ref[...],
                           preferred_element_type=jnp.float32) + bv_ref[...]
        ts_out[0] = ts_scr[...]


def _attn_body(q_ref, t_ref, mask_ref, ts_ref, k_ref, v_ref,
               dtd_ref, wqt_ref, bq_ref, wot_ref, bo_ref, out_ref):
    q = q_ref[0]
    tcol = t_ref[0]
    mcol = mask_ref[0]
    tsr = ts_ref[0]
    kk = k_ref[0]
    vv = v_ref[0]
    dt_d = dtd_ref[0, 0]

    qp = jnp.dot(q, wqt_ref[...],
                 preferred_element_type=jnp.float32) + bq_ref[...]
    delta = jnp.abs(tcol - tsr)
    wmask = jnp.logical_and(delta <= dt_d, mcol != 0.0)

    scale = 1.0 / math.sqrt(HD)
    acc = jnp.zeros((TN, D), jnp.float32) + bo_ref[...]
    for h in range(NH):
        qh = qp[:, h * HD:(h + 1) * HD]
        kh = kk[:, h * HD:(h + 1) * HD]
        vh = vv[:, h * HD:(h + 1) * HD]
        sc = jax.lax.dot_general(qh, kh, (((1,), (1,)), ((), ())),
                                 preferred_element_type=jnp.float32) * scale
        sc = jnp.where(wmask, sc, jnp.float32(-1e9))
        m = jnp.max(sc, axis=1, keepdims=True)
        p = jnp.exp(sc - m)
        attn = p / jnp.sum(p, axis=1, keepdims=True)
        oh = jnp.dot(attn, vh, preferred_element_type=jnp.float32)
        acc += jnp.dot(oh, wot_ref[pl.ds(h * HD, HD), :],
                       preferred_element_type=jnp.float32)
    out_ref[0] = acc


def kernel(q_rot, spike, time_norm, var_id, mask, params):
    B, N, Dm = q_rot.shape
    nblk = N // TN
    rows = N // 128

    wst, bs = _build_wt(params['seed'])
    wat, ba = _build_wt(params['aggr'])
    wqt, bq = _build_wt(params['fc_q'])
    wkt, bk = _build_wt(params['fc_k'])
    wvt, bv = _build_wt(params['fc_v'])
    wot, bo = _build_wt(params['fc_o'])
    dt_a = jnp.clip(jnp.exp(params['log_dt_aggr']), 0.001, 1.0).reshape(1, 1)
    dt_d = jnp.clip(jnp.exp(params['log_dt_dist']), 0.001, 1.0).reshape(1, 1)

    spike2d = spike.reshape(B, rows, 128)
    mask2d = mask.reshape(B, rows, 128)

    posq2d = pl.pallas_call(
        _select_body,
        grid=(B,),
        in_specs=[
            pl.BlockSpec((1, rows, 128), lambda b: (b, 0, 0)),
            pl.BlockSpec((1, rows, 128), lambda b: (b, 0, 0)),
        ],
        out_specs=pl.BlockSpec((1, rows, 128), lambda b: (b, 0, 0)),
        out_shape=jax.ShapeDtypeStruct((B, rows, 128), jnp.float32),
    )(spike2d, mask2d)

    posq_row = posq2d.reshape(B, 1, N)
    spike_row = spike.reshape(B, 1, N)
    t_row = time_norm.reshape(B, 1, N)
    var_row = var_id.astype(jnp.float32).reshape(B, 1, N)
    mask_row = mask.reshape(B, 1, N)

    row_spec = pl.BlockSpec((1, 1, N), lambda b, i: (b, 0, 0))
    blk_row_spec = pl.BlockSpec((1, 1, TN), lambda b, i: (b, 0, i))
    w_spec = pl.BlockSpec((D, D), lambda b, i: (0, 0))
    b_spec = pl.BlockSpec((1, D), lambda b, i: (0, 0))
    smem_spec = pl.BlockSpec(memory_space=pltpu.SMEM)

    k_ev, v_ev, ts_ev = pl.pallas_call(
        functools.partial(_events_body, nblk=nblk),
        grid=(B, nblk),
        in_specs=[
            row_spec, row_spec, row_spec,
            blk_row_spec, blk_row_spec, blk_row_spec, blk_row_spec,
            blk_row_spec,
            pl.BlockSpec((1, TN, D), lambda b, i: (b, i, 0)),
            smem_spec,
            w_spec, b_spec, w_spec, b_spec, w_spec, b_spec, w_spec, b_spec,
        ],
        out_specs=[
            pl.BlockSpec((1, KE, D), lambda b, i: (b, 0, 0)),
            pl.BlockSpec((1, KE, D), lambda b, i: (b, 0, 0)),
            pl.BlockSpec((1, KE, 1), lambda b, i: (b, 0, 0)),
        ],
        out_shape=[
            jax.ShapeDtypeStruct((B, KE, D), jnp.float32),
            jax.ShapeDtypeStruct((B, KE, D), jnp.float32),
            jax.ShapeDtypeStruct((B, KE, 1), jnp.float32),
        ],
        scratch_shapes=[
            pltpu.VMEM((KE, 1), jnp.float32),
            pltpu.VMEM((KE, 1), jnp.float32),
            pltpu.VMEM((KE, D), jnp.float32),
            pltpu.VMEM((KE, 1), jnp.float32),
            pltpu.VMEM((KE, D), jnp.float32),
        ],
    )(posq_row, t_row, var_row,
      posq_row, spike_row, t_row, var_row, mask_row, q_rot, dt_a,
      wst, bs, wat, ba, wkt, bk, wvt, bv)

    ts_row = ts_ev.reshape(B, 1, KE)
    t_col = time_norm.reshape(B, N, 1)
    mask_col = mask.reshape(B, N, 1)

    out = pl.pallas_call(
        _attn_body,
        grid=(B, nblk),
        in_specs=[
            pl.BlockSpec((1, TN, D), lambda b, i: (b, i, 0)),
            pl.BlockSpec((1, TN, 1), lambda b, i: (b, i, 0)),
            pl.BlockSpec((1, TN, 1), lambda b, i: (b, i, 0)),
            pl.BlockSpec((1, 1, KE), lambda b, i: (b, 0, 0)),
            pl.BlockSpec((1, KE, D), lambda b, i: (b, 0, 0)),
            pl.BlockSpec((1, KE, D), lambda b, i: (b, 0, 0)),
            smem_spec,
            w_spec, b_spec, w_spec, b_spec,
        ],
        out_specs=pl.BlockSpec((1, TN, D), lambda b, i: (b, i, 0)),
        out_shape=jax.ShapeDtypeStruct((B, N, Dm), jnp.float32),
    )(q_rot, t_col, mask_col, ts_row, k_ev, v_ev, dt_d, wqt, bq, wot, bo)

    return out


# speedup vs baseline: 22.0488x; 1.0047x over previous
import functools
import math

import jax
import jax.numpy as jnp
from jax import lax
from jax.experimental import pallas as pl
from jax.experimental.pallas import tpu as pltpu
from jax.experimental.pallas import tpu_sc as plsc

D = 256
KE = 128
NH = 4
HD = D // NH
TN = 512
NPART = 4


def _build_wt(p):
    r, i, j, k = p['r'], p['i'], p['j'], p['k']
    W = jnp.concatenate([
        jnp.concatenate([r, -i, -j, -k], 1),
        jnp.concatenate([i, r, -k, j], 1),
        jnp.concatenate([j, k, r, -i], 1),
        jnp.concatenate([k, -j, i, r], 1),
    ], 0)
    return W.T, p['b'].reshape(1, -1)


def _bs_tau(kref, nv):
    def outer(_, carry):
        lo, hi = carry
        mid = (lo & hi) + ((lo ^ hi) >> 1)

        def inner(i, acc):
            kv = kref[pl.ds(i * 16, 16)]
            return acc + jnp.where(kv >= mid, 1, 0).astype(jnp.int32)

        accv = lax.fori_loop(0, nv, inner, jnp.zeros((16,), jnp.int32))
        ge = jnp.sum(accv) >= KE
        return (jnp.where(ge, mid, lo), jnp.where(ge, hi, mid))

    lo, _ = lax.fori_loop(0, 32, outer,
                          (jnp.int32(-2**31), jnp.int32(2**31 - 1)))
    return lo


def _count_gt(kref, nv, tau):
    def inner(i, acc):
        kv = kref[pl.ds(i * 16, 16)]
        return acc + jnp.where(kv > tau, 1, 0).astype(jnp.int32)

    return jnp.sum(lax.fori_loop(0, nv, inner,
                                 jnp.zeros((16,), jnp.int32)))


def _compact(kref, nv, tau, r_need, write_fn):
    def body(i, carry):
        nsel, eqseen = carry
        kv = kref[pl.ds(i * 16, 16)]
        gt = kv > tau
        eq = kv == tau
        eqi = jnp.where(eq, 1, 0).astype(jnp.int32)
        eqpos = eqseen + plsc.cumsum(eqi) - eqi
        sel = gt | (eq & (eqpos < r_need))
        seli = jnp.where(sel, 1, 0).astype(jnp.int32)
        slot = nsel + plsc.cumsum(seli) - seli
        write_fn(i, slot, sel)
        return (nsel + jnp.sum(seli), eqseen + jnp.sum(eqi))

    lax.fori_loop(0, nv, body, (jnp.int32(0), jnp.int32(0)))


def _sc_select(spike_f, mask_f, t_f, var_f, B, N):
    nloc = N // NPART
    ncand = NPART * KE
    mesh = plsc.VectorSubcoreMesh(core_axis_name="c", subcore_axis_name="s")

    @functools.partial(
        pl.kernel,
        out_type=[jax.ShapeDtypeStruct((B * KE,), jnp.float32)] * 3,
        mesh=mesh,
        compiler_params=pltpu.CompilerParams(needs_layout_passes=False),
        scratch_types=[
            pltpu.VMEM((nloc,), jnp.float32),
            pltpu.VMEM((nloc,), jnp.float32),
            pltpu.VMEM((nloc,), jnp.int32),
            pltpu.VMEM((KE,), jnp.int32),
            pltpu.VMEM((KE,), jnp.int32),
            pltpu.VMEM_SHARED((B * NPART * KE,), jnp.int32),
            pltpu.VMEM_SHARED((B * NPART * KE,), jnp.int32),
            pltpu.VMEM((ncand,), jnp.int32),
            pltpu.VMEM((ncand,), jnp.int32),
            pltpu.VMEM((N,), jnp.float32),
            pltpu.VMEM((N,), jnp.float32),
            pltpu.VMEM((KE,), jnp.float32),
            pltpu.VMEM((KE,), jnp.float32),
            pltpu.VMEM((KE,), jnp.float32),
        ],
    )
    def sel(spike_h, mask_h, t_h, var_h, oidx_h, ots_h, ovs_h,
            sp_v, mk_v, key_v, ck_v, ci_v, shk, shi, mgk_v, mgi_v,
            tv, vv, oi_v, ot_v, ov_v):
        cid = lax.axis_index("c")
        sid = lax.axis_index("s")

        @pl.when(cid == 0)
        def _core0():
            row = sid // NPART
            part = sid % NPART
            base = row * N + part * nloc
            pltpu.sync_copy(spike_h.at[pl.ds(base, nloc)], sp_v)
            pltpu.sync_copy(mask_h.at[pl.ds(base, nloc)], mk_v)

            def keys_body(i, _):
                s = sp_v[pl.ds(i * 16, 16)] * mk_v[pl.ds(i * 16, 16)]
                b0 = lax.bitcast_convert_type(s, jnp.int32)
                key_v[pl.ds(i * 16, 16)] = b0 ^ (
                    lax.shift_right_arithmetic(b0, 31) & jnp.int32(0x7FFFFFFF))
                return 0

            lax.fori_loop(0, nloc // 16, keys_body, 0)

            tau1 = _bs_tau(key_v, nloc // 16)
            rn1 = KE - _count_gt(key_v, nloc // 16, tau1)

            def wr1(i, slot, selm):
                kv = key_v[pl.ds(i * 16, 16)]
                gidx = (part * nloc + i * 16
                        + lax.broadcasted_iota(jnp.int32, (16,), 0))
                plsc.store_scatter(ck_v, [slot], kv, mask=selm)
                plsc.store_scatter(ci_v, [slot], gidx, mask=selm)

            _compact(key_v, nloc // 16, tau1, rn1, wr1)

            pltpu.sync_copy(ck_v, shk.at[pl.ds(sid * KE, KE)])
            pltpu.sync_copy(ci_v, shi.at[pl.ds(sid * KE, KE)])
            plsc.subcore_barrier()

            @pl.when(sid < B)
            def _merge():
                pltpu.sync_copy(shk.at[pl.ds(sid * ncand, ncand)], mgk_v)
                pltpu.sync_copy(shi.at[pl.ds(sid * ncand, ncand)], mgi_v)
                pltpu.sync_copy(t_h.at[pl.ds(sid * N, N)], tv)
                pltpu.sync_copy(var_h.at[pl.ds(sid * N, N)], vv)

                tau2 = _bs_tau(mgk_v, ncand // 16)
                rn2 = KE - _count_gt(mgk_v, ncand // 16, tau2)

                def wr2(i, slot, selm):
                    iv = mgi_v[pl.ds(i * 16, 16)]
                    tg = plsc.load_gather(tv, [iv])
                    vg = plsc.load_gather(vv, [iv])
                    plsc.store_scatter(oi_v, [slot],
                                       iv.astype(jnp.float32), mask=selm)
                    plsc.store_scatter(ot_v, [slot], tg, mask=selm)
                    plsc.store_scatter(ov_v, [slot], vg, mask=selm)

                _compact(mgk_v, ncand // 16, tau2, rn2, wr2)

                pltpu.sync_copy(oi_v, oidx_h.at[pl.ds(sid * KE, KE)])
                pltpu.sync_copy(ot_v, ots_h.at[pl.ds(sid * KE, KE)])
                pltpu.sync_copy(ov_v, ovs_h.at[pl.ds(sid * KE, KE)])

    return sel(spike_f, mask_f, t_f, var_f)


def _events_body(idx_ref, ts_ref, vs_ref, sb_ref, tb_ref, vb_ref, mb_ref,
                 q_ref, dta_ref, wst_ref, bs_ref, wat_ref, ba_ref,
                 wkt_ref, bk_ref, wvt_ref, bv_ref,
                 k_out, v_out,
                 num_scr, den_scr, qseed_scr, *, nblk):
    i = pl.program_id(1)
    q = q_ref[0]
    dt_a = dta_ref[0, 0]

    @pl.when(i == 0)
    def _init():
        num_scr[...] = jnp.zeros_like(num_scr)
        den_scr[...] = jnp.zeros_like(den_scr)
        qseed_scr[...] = jnp.zeros_like(qseed_scr)

    idxc = idx_ref[0]
    tsc = ts_ref[0]
    vsc = vs_ref[0]
    tb = tb_ref[0]
    vb = vb_ref[0]
    sb = sb_ref[0]
    mb = mb_ref[0]

    pos = (jax.lax.broadcasted_iota(jnp.int32, (KE, TN), 1)
           + i * TN).astype(jnp.float32)
    s_blk = (idxc == pos).astype(jnp.float32)
    qseed_scr[...] += jax.lax.dot_general(
        s_blk, q, (((1,), (0,)), ((), ())), preferred_element_type=jnp.float32)

    delta = jnp.abs(tb - tsc)
    win = (delta <= dt_a).astype(jnp.float32)
    decay = jnp.exp(-2.0 * delta / jnp.maximum(dt_a, 0.001))
    vaff = 0.5 + 0.5 * (vb == vsc).astype(jnp.float32)
    incid = win * decay * vaff * sb * mb
    num_scr[...] += jax.lax.dot_general(
        incid, q, (((1,), (0,)), ((), ())), preferred_element_type=jnp.float32)
    den_scr[...] += jnp.sum(incid, axis=1, keepdims=True)

    @pl.when(i == nblk - 1)
    def _fin():
        hs = jnp.dot(qseed_scr[...], wst_ref[...],
                     preferred_element_type=jnp.float32) + bs_ref[...]
        agg = num_scr[...] / jnp.maximum(den_scr[...], 1e-6)
        he = hs + jnp.dot(agg, wat_ref[...],
                          preferred_element_type=jnp.float32) + ba_ref[...]
        k_out[0] = jnp.dot(he, wkt_ref[...],
                           preferred_element_type=jnp.float32) + bk_ref[...]
        v_out[0] = jnp.dot(he, wvt_ref[...],
                           preferred_element_type=jnp.float32) + bv_ref[...]


def _attn_body(q_ref, t_ref, mask_ref, ts_ref, k_ref, v_ref,
               dtd_ref, wqt_ref, bq_ref, wot_ref, bo_ref, out_ref):
    q = q_ref[0]
    tcol = t_ref[0]
    mcol = mask_ref[0]
    tsr = ts_ref[0]
    kk = k_ref[0]
    vv = v_ref[0]
    dt_d = dtd_ref[0, 0]

    qp = jnp.dot(q, wqt_ref[...],
                 preferred_element_type=jnp.float32) + bq_ref[...]
    delta = jnp.abs(tcol - tsr)
    wmask = jnp.logical_and(delta <= dt_d, mcol != 0.0)

    scale = 1.0 / math.sqrt(HD)
    acc = jnp.zeros((TN, D), jnp.float32) + bo_ref[...]
    for h in range(NH):
        qh = qp[:, h * HD:(h + 1) * HD]
        kh = kk[:, h * HD:(h + 1) * HD]
        vh = vv[:, h * HD:(h + 1) * HD]
        sc = jax.lax.dot_general(qh, kh, (((1,), (1,)), ((), ())),
                                 preferred_element_type=jnp.float32) * scale
        sc = jnp.where(wmask, sc, jnp.float32(-1e9))
        m = jnp.max(sc, axis=1, keepdims=True)
        p = jnp.exp(sc - m)
        attn = p / jnp.sum(p, axis=1, keepdims=True)
        oh = jnp.dot(attn, vh, preferred_element_type=jnp.float32)
        acc += jnp.dot(oh, wot_ref[pl.ds(h * HD, HD), :],
                       preferred_element_type=jnp.float32)
    out_ref[0] = acc


def kernel(q_rot, spike, time_norm, var_id, mask, params):
    B, N, Dm = q_rot.shape
    nblk = N // TN

    wst, bs = _build_wt(params['seed'])
    wat, ba = _build_wt(params['aggr'])
    wqt, bq = _build_wt(params['fc_q'])
    wkt, bk = _build_wt(params['fc_k'])
    wvt, bv = _build_wt(params['fc_v'])
    wot, bo = _build_wt(params['fc_o'])
    dt_a = jnp.clip(jnp.exp(params['log_dt_aggr']), 0.001, 1.0).reshape(1, 1)
    dt_d = jnp.clip(jnp.exp(params['log_dt_dist']), 0.001, 1.0).reshape(1, 1)

    var_f = var_id.astype(jnp.float32)
    idxf, tsf, vsf = _sc_select(
        spike.reshape(B * N), mask.reshape(B * N),
        time_norm.reshape(B * N), var_f.reshape(B * N), B, N)

    idx_col = idxf.reshape(B, KE, 1)
    ts_col = tsf.reshape(B, KE, 1)
    vs_col = vsf.reshape(B, KE, 1)

    spike_row = spike.reshape(B, 1, N)
    t_row = time_norm.reshape(B, 1, N)
    var_row = var_f.reshape(B, 1, N)
    mask_row = mask.reshape(B, 1, N)

    col_spec = pl.BlockSpec((1, KE, 1), lambda b, i: (b, 0, 0))
    blk_row_spec = pl.BlockSpec((1, 1, TN), lambda b, i: (b, 0, i))
    w_spec = pl.BlockSpec((D, D), lambda b, i: (0, 0))
    b_spec = pl.BlockSpec((1, D), lambda b, i: (0, 0))
    smem_spec = pl.BlockSpec(memory_space=pltpu.SMEM)

    k_ev, v_ev = pl.pallas_call(
        functools.partial(_events_body, nblk=nblk),
        grid=(B, nblk),
        in_specs=[
            col_spec, col_spec, col_spec,
            blk_row_spec, blk_row_spec, blk_row_spec, blk_row_spec,
            pl.BlockSpec((1, TN, D), lambda b, i: (b, i, 0)),
            smem_spec,
            w_spec, b_spec, w_spec, b_spec, w_spec, b_spec, w_spec, b_spec,
        ],
        out_specs=[
            pl.BlockSpec((1, KE, D), lambda b, i: (b, 0, 0)),
            pl.BlockSpec((1, KE, D), lambda b, i: (b, 0, 0)),
        ],
        out_shape=[
            jax.ShapeDtypeStruct((B, KE, D), jnp.float32),
            jax.ShapeDtypeStruct((B, KE, D), jnp.float32),
        ],
        scratch_shapes=[
            pltpu.VMEM((KE, D), jnp.float32),
            pltpu.VMEM((KE, 1), jnp.float32),
            pltpu.VMEM((KE, D), jnp.float32),
        ],
    )(idx_col, ts_col, vs_col,
      spike_row, t_row, var_row, mask_row, q_rot, dt_a,
      wst, bs, wat, ba, wkt, bk, wvt, bv)

    ts_row = tsf.reshape(B, 1, KE)
    t_col = time_norm.reshape(B, N, 1)
    mask_col = mask.reshape(B, N, 1)

    out = pl.pallas_call(
        _attn_body,
        grid=(B, nblk),
        in_specs=[
            pl.BlockSpec((1, TN, D), lambda b, i: (b, i, 0)),
            pl.BlockSpec((1, TN, 1), lambda b, i: (b, i, 0)),
            pl.BlockSpec((1, TN, 1), lambda b, i: (b, i, 0)),
            pl.BlockSpec((1, 1, KE), lambda b, i: (b, 0, 0)),
            pl.BlockSpec((1, KE, D), lambda b, i: (b, 0, 0)),
            pl.BlockSpec((1, KE, D), lambda b, i: (b, 0, 0)),
            smem_spec,
            w_spec, b_spec, w_spec, b_spec,
        ],
        out_specs=pl.BlockSpec((1, TN, D), lambda b, i: (b, i, 0)),
        out_shape=jax.ShapeDtypeStruct((B, N, Dm), jnp.float32),
    )(q_rot, t_col, mask_col, ts_row, k_ev, v_ev, dt_d, wqt, bq, wot, bo)

    return out


# speedup vs baseline: 24.6638x; 1.1186x over previous
import functools
import math

import jax
import jax.numpy as jnp
from jax import lax
from jax.experimental import pallas as pl
from jax.experimental.pallas import tpu as pltpu
from jax.experimental.pallas import tpu_sc as plsc

D = 256
KE = 128
NH = 4
HD = D // NH
TN = 1024
NPART = 4


def _build_wt(p):
    r, i, j, k = p['r'], p['i'], p['j'], p['k']
    W = jnp.concatenate([
        jnp.concatenate([r, -i, -j, -k], 1),
        jnp.concatenate([i, r, -k, j], 1),
        jnp.concatenate([j, k, r, -i], 1),
        jnp.concatenate([k, -j, i, r], 1),
    ], 0)
    return W.T, p['b'].reshape(1, -1)


def _bs_tau(kref, nv):
    def outer(_, carry):
        lo, hi = carry
        mid = (lo & hi) + ((lo ^ hi) >> 1)

        def inner(i, acc):
            kv = kref[pl.ds(i * 16, 16)]
            return acc + jnp.where(kv >= mid, 1, 0).astype(jnp.int32)

        accv = lax.fori_loop(0, nv, inner, jnp.zeros((16,), jnp.int32))
        ge = jnp.sum(accv) >= KE
        return (jnp.where(ge, mid, lo), jnp.where(ge, hi, mid))

    lo, _ = lax.fori_loop(0, 32, outer,
                          (jnp.int32(-2**31), jnp.int32(2**31 - 1)))
    return lo


def _count_gt(kref, nv, tau):
    def inner(i, acc):
        kv = kref[pl.ds(i * 16, 16)]
        return acc + jnp.where(kv > tau, 1, 0).astype(jnp.int32)

    return jnp.sum(lax.fori_loop(0, nv, inner,
                                 jnp.zeros((16,), jnp.int32)))


def _compact(kref, nv, tau, r_need, write_fn):
    def body(i, carry):
        nsel, eqseen = carry
        kv = kref[pl.ds(i * 16, 16)]
        gt = kv > tau
        eq = kv == tau
        eqi = jnp.where(eq, 1, 0).astype(jnp.int32)
        eqpos = eqseen + plsc.cumsum(eqi) - eqi
        sel = gt | (eq & (eqpos < r_need))
        seli = jnp.where(sel, 1, 0).astype(jnp.int32)
        slot = nsel + plsc.cumsum(seli) - seli
        write_fn(i, slot, sel)
        return (nsel + jnp.sum(seli), eqseen + jnp.sum(eqi))

    lax.fori_loop(0, nv, body, (jnp.int32(0), jnp.int32(0)))


def _sc_select(spike_f, mask_f, t_f, var_f, B, N):
    nloc = N // NPART
    ncand = NPART * KE
    mesh = plsc.VectorSubcoreMesh(core_axis_name="c", subcore_axis_name="s")

    @functools.partial(
        pl.kernel,
        out_type=[jax.ShapeDtypeStruct((B * KE,), jnp.float32)] * 3,
        mesh=mesh,
        compiler_params=pltpu.CompilerParams(needs_layout_passes=False),
        scratch_types=[
            pltpu.VMEM((nloc,), jnp.float32),
            pltpu.VMEM((nloc,), jnp.float32),
            pltpu.VMEM((nloc,), jnp.int32),
            pltpu.VMEM((KE,), jnp.int32),
            pltpu.VMEM((KE,), jnp.int32),
            pltpu.VMEM_SHARED((B * NPART * KE,), jnp.int32),
            pltpu.VMEM_SHARED((B * NPART * KE,), jnp.int32),
            pltpu.VMEM((ncand,), jnp.int32),
            pltpu.VMEM((ncand,), jnp.int32),
            pltpu.VMEM((N,), jnp.float32),
            pltpu.VMEM((N,), jnp.float32),
            pltpu.VMEM((KE,), jnp.float32),
            pltpu.VMEM((KE,), jnp.float32),
            pltpu.VMEM((KE,), jnp.float32),
        ],
    )
    def sel(spike_h, mask_h, t_h, var_h, oidx_h, ots_h, ovs_h,
            sp_v, mk_v, key_v, ck_v, ci_v, shk, shi, mgk_v, mgi_v,
            tv, vv, oi_v, ot_v, ov_v):
        cid = lax.axis_index("c")
        sid = lax.axis_index("s")

        @pl.when(cid == 0)
        def _core0():
            row = sid // NPART
            part = sid % NPART
            base = row * N + part * nloc
            pltpu.sync_copy(spike_h.at[pl.ds(base, nloc)], sp_v)
            pltpu.sync_copy(mask_h.at[pl.ds(base, nloc)], mk_v)

            def keys_body(i, _):
                s = sp_v[pl.ds(i * 16, 16)] * mk_v[pl.ds(i * 16, 16)]
                b0 = lax.bitcast_convert_type(s, jnp.int32)
                key_v[pl.ds(i * 16, 16)] = b0 ^ (
                    lax.shift_right_arithmetic(b0, 31) & jnp.int32(0x7FFFFFFF))
                return 0

            lax.fori_loop(0, nloc // 16, keys_body, 0)

            tau1 = _bs_tau(key_v, nloc // 16)
            rn1 = KE - _count_gt(key_v, nloc // 16, tau1)

            def wr1(i, slot, selm):
                kv = key_v[pl.ds(i * 16, 16)]
                gidx = (part * nloc + i * 16
                        + lax.broadcasted_iota(jnp.int32, (16,), 0))
                plsc.store_scatter(ck_v, [slot], kv, mask=selm)
                plsc.store_scatter(ci_v, [slot], gidx, mask=selm)

            _compact(key_v, nloc // 16, tau1, rn1, wr1)

            pltpu.sync_copy(ck_v, shk.at[pl.ds(sid * KE, KE)])
            pltpu.sync_copy(ci_v, shi.at[pl.ds(sid * KE, KE)])
            plsc.subcore_barrier()

            @pl.when(sid < B)
            def _merge():
                pltpu.sync_copy(shk.at[pl.ds(sid * ncand, ncand)], mgk_v)
                pltpu.sync_copy(shi.at[pl.ds(sid * ncand, ncand)], mgi_v)
                pltpu.sync_copy(t_h.at[pl.ds(sid * N, N)], tv)
                pltpu.sync_copy(var_h.at[pl.ds(sid * N, N)], vv)

                tau2 = _bs_tau(mgk_v, ncand // 16)
                rn2 = KE - _count_gt(mgk_v, ncand // 16, tau2)

                def wr2(i, slot, selm):
                    iv = mgi_v[pl.ds(i * 16, 16)]
                    tg = plsc.load_gather(tv, [iv])
                    vg = plsc.load_gather(vv, [iv])
                    plsc.store_scatter(oi_v, [slot],
                                       iv.astype(jnp.float32), mask=selm)
                    plsc.store_scatter(ot_v, [slot], tg, mask=selm)
                    plsc.store_scatter(ov_v, [slot], vg, mask=selm)

                _compact(mgk_v, ncand // 16, tau2, rn2, wr2)

                pltpu.sync_copy(oi_v, oidx_h.at[pl.ds(sid * KE, KE)])
                pltpu.sync_copy(ot_v, ots_h.at[pl.ds(sid * KE, KE)])
                pltpu.sync_copy(ov_v, ovs_h.at[pl.ds(sid * KE, KE)])

    return sel(spike_f, mask_f, t_f, var_f)


def _events_body(idx_ref, ts_ref, vs_ref, sb_ref, tb_ref, vb_ref, mb_ref,
                 q_ref, dta_ref, wst_ref, bs_ref, wat_ref, ba_ref,
                 wkt_ref, bk_ref, wvt_ref, bv_ref,
                 k_out, v_out,
                 num_scr, den_scr, qseed_scr, *, nblk):
    i = pl.program_id(1)
    q = q_ref[0]
    dt_a = dta_ref[0, 0]

    @pl.when(i == 0)
    def _init():
        num_scr[...] = jnp.zeros_like(num_scr)
        den_scr[...] = jnp.zeros_like(den_scr)
        qseed_scr[...] = jnp.zeros_like(qseed_scr)

    idxc = idx_ref[0]
    tsc = ts_ref[0]
    vsc = vs_ref[0]
    tb = tb_ref[0]
    vb = vb_ref[0]
    sb = sb_ref[0]
    mb = mb_ref[0]

    pos = (jax.lax.broadcasted_iota(jnp.int32, (KE, TN), 1)
           + i * TN).astype(jnp.float32)
    s_blk = (idxc == pos).astype(jnp.float32)
    qseed_scr[...] += jax.lax.dot_general(
        s_blk, q, (((1,), (0,)), ((), ())), preferred_element_type=jnp.float32)

    delta = jnp.abs(tb - tsc)
    win = (delta <= dt_a).astype(jnp.float32)
    decay = jnp.exp(-2.0 * delta / jnp.maximum(dt_a, 0.001))
    vaff = 0.5 + 0.5 * (vb == vsc).astype(jnp.float32)
    incid = win * decay * vaff * sb * mb
    num_scr[...] += jax.lax.dot_general(
        incid, q, (((1,), (0,)), ((), ())), preferred_element_type=jnp.float32)
    den_scr[...] += jnp.sum(incid, axis=1, keepdims=True)

    @pl.when(i == nblk - 1)
    def _fin():
        hs = jnp.dot(qseed_scr[...], wst_ref[...],
                     preferred_element_type=jnp.float32) + bs_ref[...]
        agg = num_scr[...] / jnp.maximum(den_scr[...], 1e-6)
        he = hs + jnp.dot(agg, wat_ref[...],
                          preferred_element_type=jnp.float32) + ba_ref[...]
        k_out[0] = jnp.dot(he, wkt_ref[...],
                           preferred_element_type=jnp.float32) + bk_ref[...]
        v_out[0] = jnp.dot(he, wvt_ref[...],
                           preferred_element_type=jnp.float32) + bv_ref[...]


def _attn_body(q_ref, t_ref, mask_ref, ts_ref, k_ref, v_ref,
               dtd_ref, wqt_ref, bq_ref, wot_ref, bo_ref, out_ref):
    q = q_ref[0]
    tcol = t_ref[0]
    mcol = mask_ref[0]
    tsr = ts_ref[0]
    kk = k_ref[0]
    vv = v_ref[0]
    dt_d = dtd_ref[0, 0]

    qp = jnp.dot(q, wqt_ref[...],
                 preferred_element_type=jnp.float32) + bq_ref[...]
    delta = jnp.abs(tcol - tsr)
    wmask = jnp.logical_and(delta <= dt_d, mcol != 0.0)

    scale = 1.0 / math.sqrt(HD)
    ohs = []
    for h in range(NH):
        qh = qp[:, h * HD:(h + 1) * HD]
        kh = kk[:, h * HD:(h + 1) * HD]
        vh = vv[:, h * HD:(h + 1) * HD]
        sc = jax.lax.dot_general(qh, kh, (((1,), (1,)), ((), ())),
                                 preferred_element_type=jnp.float32) * scale
        sc = jnp.where(wmask, sc, jnp.float32(-1e9))
        m = jnp.max(sc, axis=1, keepdims=True)
        p = jnp.exp(sc - m)
        rs = 1.0 / jnp.sum(p, axis=1, keepdims=True)
        ohs.append(jnp.dot(p, vh, preferred_element_type=jnp.float32) * rs)
    out_ref[0] = jnp.dot(jnp.concatenate(ohs, 1), wot_ref[...],
                         preferred_element_type=jnp.float32) + bo_ref[...]


def kernel(q_rot, spike, time_norm, var_id, mask, params):
    B, N, Dm = q_rot.shape
    nblk = N // TN

    wst, bs = _build_wt(params['seed'])
    wat, ba = _build_wt(params['aggr'])
    wqt, bq = _build_wt(params['fc_q'])
    wkt, bk = _build_wt(params['fc_k'])
    wvt, bv = _build_wt(params['fc_v'])
    wot, bo = _build_wt(params['fc_o'])
    dt_a = jnp.clip(jnp.exp(params['log_dt_aggr']), 0.001, 1.0).reshape(1, 1)
    dt_d = jnp.clip(jnp.exp(params['log_dt_dist']), 0.001, 1.0).reshape(1, 1)

    var_f = var_id.astype(jnp.float32)
    idxf, tsf, vsf = _sc_select(
        spike.reshape(B * N), mask.reshape(B * N),
        time_norm.reshape(B * N), var_f.reshape(B * N), B, N)

    idx_col = idxf.reshape(B, KE, 1)
    ts_col = tsf.reshape(B, KE, 1)
    vs_col = vsf.reshape(B, KE, 1)

    spike_row = spike.reshape(B, 1, N)
    t_row = time_norm.reshape(B, 1, N)
    var_row = var_f.reshape(B, 1, N)
    mask_row = mask.reshape(B, 1, N)

    col_spec = pl.BlockSpec((1, KE, 1), lambda b, i: (b, 0, 0))
    blk_row_spec = pl.BlockSpec((1, 1, TN), lambda b, i: (b, 0, i))
    w_spec = pl.BlockSpec((D, D), lambda b, i: (0, 0))
    b_spec = pl.BlockSpec((1, D), lambda b, i: (0, 0))
    smem_spec = pl.BlockSpec(memory_space=pltpu.SMEM)

    k_ev, v_ev = pl.pallas_call(
        functools.partial(_events_body, nblk=nblk),
        grid=(B, nblk),
        in_specs=[
            col_spec, col_spec, col_spec,
            blk_row_spec, blk_row_spec, blk_row_spec, blk_row_spec,
            pl.BlockSpec((1, TN, D), lambda b, i: (b, i, 0)),
            smem_spec,
            w_spec, b_spec, w_spec, b_spec, w_spec, b_spec, w_spec, b_spec,
        ],
        out_specs=[
            pl.BlockSpec((1, KE, D), lambda b, i: (b, 0, 0)),
            pl.BlockSpec((1, KE, D), lambda b, i: (b, 0, 0)),
        ],
        out_shape=[
            jax.ShapeDtypeStruct((B, KE, D), jnp.float32),
            jax.ShapeDtypeStruct((B, KE, D), jnp.float32),
        ],
        scratch_shapes=[
            pltpu.VMEM((KE, D), jnp.float32),
            pltpu.VMEM((KE, 1), jnp.float32),
            pltpu.VMEM((KE, D), jnp.float32),
        ],
    )(idx_col, ts_col, vs_col,
      spike_row, t_row, var_row, mask_row, q_rot, dt_a,
      wst, bs, wat, ba, wkt, bk, wvt, bv)

    ts_row = tsf.reshape(B, 1, KE)
    t_col = time_norm.reshape(B, N, 1)
    mask_col = mask.reshape(B, N, 1)

    out = pl.pallas_call(
        _attn_body,
        grid=(B, nblk),
        in_specs=[
            pl.BlockSpec((1, TN, D), lambda b, i: (b, i, 0)),
            pl.BlockSpec((1, TN, 1), lambda b, i: (b, i, 0)),
            pl.BlockSpec((1, TN, 1), lambda b, i: (b, i, 0)),
            pl.BlockSpec((1, 1, KE), lambda b, i: (b, 0, 0)),
            pl.BlockSpec((1, KE, D), lambda b, i: (b, 0, 0)),
            pl.BlockSpec((1, KE, D), lambda b, i: (b, 0, 0)),
            smem_spec,
            w_spec, b_spec, w_spec, b_spec,
        ],
        out_specs=pl.BlockSpec((1, TN, D), lambda b, i: (b, i, 0)),
        out_shape=jax.ShapeDtypeStruct((B, N, Dm), jnp.float32),
    )(q_rot, t_col, mask_col, ts_row, k_ev, v_ev, dt_d, wqt, bq, wot, bo)

    return out


# speedup vs baseline: 35.5524x; 1.4415x over previous
import functools
import math

import jax
import jax.numpy as jnp
from jax import lax
from jax.experimental import pallas as pl
from jax.experimental.pallas import tpu as pltpu
from jax.experimental.pallas import tpu_sc as plsc

D = 256
KE = 128
NH = 4
HD = D // NH
TN = 1024
NPART = 4


def _build_wt(p):
    r, i, j, k = p['r'], p['i'], p['j'], p['k']
    W = jnp.concatenate([
        jnp.concatenate([r, -i, -j, -k], 1),
        jnp.concatenate([i, r, -k, j], 1),
        jnp.concatenate([j, k, r, -i], 1),
        jnp.concatenate([k, -j, i, r], 1),
    ], 0)
    return W.T, p['b'].reshape(1, -1)


def _bs_tau(kref, nv):
    def outer(_, carry):
        lo, hi = carry
        mid = (lo & hi) + ((lo ^ hi) >> 1)

        def inner(i, acc):
            kv = kref[pl.ds(i * 16, 16)]
            return acc + jnp.where(kv >= mid, 1, 0).astype(jnp.int32)

        accv = lax.fori_loop(0, nv, inner, jnp.zeros((16,), jnp.int32))
        ge = jnp.sum(accv) >= KE
        return (jnp.where(ge, mid, lo), jnp.where(ge, hi, mid))

    lo, _ = lax.fori_loop(0, 32, outer,
                          (jnp.int32(-2**31), jnp.int32(2**31 - 1)))
    return lo


def _count_gt(kref, nv, tau):
    def inner(i, acc):
        kv = kref[pl.ds(i * 16, 16)]
        return acc + jnp.where(kv > tau, 1, 0).astype(jnp.int32)

    return jnp.sum(lax.fori_loop(0, nv, inner,
                                 jnp.zeros((16,), jnp.int32)))


def _compact(kref, nv, tau, r_need, write_fn):
    def body(i, carry):
        nsel, eqseen = carry
        kv = kref[pl.ds(i * 16, 16)]
        gt = kv > tau
        eq = kv == tau
        eqi = jnp.where(eq, 1, 0).astype(jnp.int32)
        eqpos = eqseen + plsc.cumsum(eqi) - eqi
        sel = gt | (eq & (eqpos < r_need))
        seli = jnp.where(sel, 1, 0).astype(jnp.int32)
        slot = nsel + plsc.cumsum(seli) - seli
        write_fn(i, slot, sel)
        return (nsel + jnp.sum(seli), eqseen + jnp.sum(eqi))

    lax.fori_loop(0, nv, body, (jnp.int32(0), jnp.int32(0)))


def _sc_select(spike_f, mask_f, t_f, var_f, B, N):
    nloc = N // NPART
    ncand = NPART * KE
    mesh = plsc.VectorSubcoreMesh(core_axis_name="c", subcore_axis_name="s")

    @functools.partial(
        pl.kernel,
        out_type=[jax.ShapeDtypeStruct((B * KE,), jnp.float32)] * 3,
        mesh=mesh,
        compiler_params=pltpu.CompilerParams(needs_layout_passes=False),
        scratch_types=[
            pltpu.VMEM((nloc,), jnp.float32),
            pltpu.VMEM((nloc,), jnp.float32),
            pltpu.VMEM((nloc,), jnp.int32),
            pltpu.VMEM((KE,), jnp.int32),
            pltpu.VMEM((KE,), jnp.int32),
            pltpu.VMEM_SHARED((B * NPART * KE,), jnp.int32),
            pltpu.VMEM_SHARED((B * NPART * KE,), jnp.int32),
            pltpu.VMEM((ncand,), jnp.int32),
            pltpu.VMEM((ncand,), jnp.int32),
            pltpu.VMEM((N,), jnp.float32),
            pltpu.VMEM((N,), jnp.float32),
            pltpu.VMEM((KE,), jnp.float32),
            pltpu.VMEM((KE,), jnp.float32),
            pltpu.VMEM((KE,), jnp.float32),
        ],
    )
    def sel(spike_h, mask_h, t_h, var_h, oidx_h, ots_h, ovs_h,
            sp_v, mk_v, key_v, ck_v, ci_v, shk, shi, mgk_v, mgi_v,
            tv, vv, oi_v, ot_v, ov_v):
        cid = lax.axis_index("c")
        sid = lax.axis_index("s")

        @pl.when(cid == 0)
        def _core0():
            row = sid // NPART
            part = sid % NPART
            base = row * N + part * nloc
            pltpu.sync_copy(spike_h.at[pl.ds(base, nloc)], sp_v)
            pltpu.sync_copy(mask_h.at[pl.ds(base, nloc)], mk_v)

            def keys_body(i, _):
                s = sp_v[pl.ds(i * 16, 16)] * mk_v[pl.ds(i * 16, 16)]
                b0 = lax.bitcast_convert_type(s, jnp.int32)
                key_v[pl.ds(i * 16, 16)] = b0 ^ (
                    lax.shift_right_arithmetic(b0, 31) & jnp.int32(0x7FFFFFFF))
                return 0

            lax.fori_loop(0, nloc // 16, keys_body, 0)

            tau1 = _bs_tau(key_v, nloc // 16)
            rn1 = KE - _count_gt(key_v, nloc // 16, tau1)

            def wr1(i, slot, selm):
                kv = key_v[pl.ds(i * 16, 16)]
                gidx = (part * nloc + i * 16
                        + lax.broadcasted_iota(jnp.int32, (16,), 0))
                plsc.store_scatter(ck_v, [slot], kv, mask=selm)
                plsc.store_scatter(ci_v, [slot], gidx, mask=selm)

            _compact(key_v, nloc // 16, tau1, rn1, wr1)

            pltpu.sync_copy(ck_v, shk.at[pl.ds(sid * KE, KE)])
            pltpu.sync_copy(ci_v, shi.at[pl.ds(sid * KE, KE)])
            plsc.subcore_barrier()

            @pl.when(sid < B)
            def _merge():
                pltpu.sync_copy(shk.at[pl.ds(sid * ncand, ncand)], mgk_v)
                pltpu.sync_copy(shi.at[pl.ds(sid * ncand, ncand)], mgi_v)
                pltpu.sync_copy(t_h.at[pl.ds(sid * N, N)], tv)
                pltpu.sync_copy(var_h.at[pl.ds(sid * N, N)], vv)

                tau2 = _bs_tau(mgk_v, ncand // 16)
                rn2 = KE - _count_gt(mgk_v, ncand // 16, tau2)

                def wr2(i, slot, selm):
                    iv = mgi_v[pl.ds(i * 16, 16)]
                    tg = plsc.load_gather(tv, [iv])
                    vg = plsc.load_gather(vv, [iv])
                    plsc.store_scatter(oi_v, [slot],
                                       iv.astype(jnp.float32), mask=selm)
                    plsc.store_scatter(ot_v, [slot], tg, mask=selm)
                    plsc.store_scatter(ov_v, [slot], vg, mask=selm)

                _compact(mgk_v, ncand // 16, tau2, rn2, wr2)

                pltpu.sync_copy(oi_v, oidx_h.at[pl.ds(sid * KE, KE)])
                pltpu.sync_copy(ot_v, ots_h.at[pl.ds(sid * KE, KE)])
                pltpu.sync_copy(ov_v, ovs_h.at[pl.ds(sid * KE, KE)])

    return sel(spike_f, mask_f, t_f, var_f)


def _events_body(idx_ref, ts_ref, vs_ref, sb_ref, tb_ref, vb_ref, mb_ref,
                 q_ref, dta_ref, wst_ref, bs_ref, wat_ref, ba_ref,
                 wkt_ref, bk_ref, wvt_ref, bv_ref,
                 wqt_ref, bq_ref, wot_ref,
                 kc_out, vo_out, bsc_out,
                 num_scr, den_scr, qseed_scr, *, nblk):
    i = pl.program_id(1)
    q = q_ref[0]
    dt_a = dta_ref[0, 0]

    @pl.when(i == 0)
    def _init():
        num_scr[...] = jnp.zeros_like(num_scr)
        den_scr[...] = jnp.zeros_like(den_scr)
        qseed_scr[...] = jnp.zeros_like(qseed_scr)

    idxc = idx_ref[0]
    tsc = ts_ref[0]
    vsc = vs_ref[0]
    tb = tb_ref[0]
    vb = vb_ref[0]
    sb = sb_ref[0]
    mb = mb_ref[0]

    pos = (jax.lax.broadcasted_iota(jnp.int32, (KE, TN), 1)
           + i * TN).astype(jnp.float32)
    s_blk = (idxc == pos).astype(jnp.float32)
    qseed_scr[...] += jax.lax.dot_general(
        s_blk, q, (((1,), (0,)), ((), ())), preferred_element_type=jnp.float32)

    delta = jnp.abs(tb - tsc)
    win = (delta <= dt_a).astype(jnp.float32)
    decay = jnp.exp(-2.0 * delta / jnp.maximum(dt_a, 0.001))
    vaff = 0.5 + 0.5 * (vb == vsc).astype(jnp.float32)
    incid = win * decay * vaff * sb * mb
    num_scr[...] += jax.lax.dot_general(
        incid, q, (((1,), (0,)), ((), ())), preferred_element_type=jnp.float32)
    den_scr[...] += jnp.sum(incid, axis=1, keepdims=True)

    @pl.when(i == nblk - 1)
    def _fin():
        hs = jnp.dot(qseed_scr[...], wst_ref[...],
                     preferred_element_type=jnp.float32) + bs_ref[...]
        agg = num_scr[...] / jnp.maximum(den_scr[...], 1e-6)
        he = hs + jnp.dot(agg, wat_ref[...],
                          preferred_element_type=jnp.float32) + ba_ref[...]
        kk = jnp.dot(he, wkt_ref[...],
                     preferred_element_type=jnp.float32) + bk_ref[...]
        vv = jnp.dot(he, wvt_ref[...],
                     preferred_element_type=jnp.float32) + bv_ref[...]
        kcs, vos, bscs = [], [], []
        for h in range(NH):
            kh = kk[:, h * HD:(h + 1) * HD]
            vh = vv[:, h * HD:(h + 1) * HD]
            kcs.append(jax.lax.dot_general(
                wqt_ref[:, pl.ds(h * HD, HD)], kh, (((1,), (1,)), ((), ())),
                preferred_element_type=jnp.float32))
            bscs.append(jax.lax.dot_general(
                bq_ref[:, pl.ds(h * HD, HD)], kh, (((1,), (1,)), ((), ())),
                preferred_element_type=jnp.float32))
            vos.append(jnp.dot(vh, wot_ref[pl.ds(h * HD, HD), :],
                               preferred_element_type=jnp.float32))
        kc_out[0] = jnp.concatenate(kcs, 1)
        vo_out[0] = jnp.concatenate(vos, 0)
        bsc_out[0] = jnp.concatenate(bscs, 1)


def _attn_body(q_ref, t_ref, mask_ref, ts4_ref, kc_ref, vo_ref, bsc_ref,
               dtd_ref, bo_ref, out_ref):
    q = q_ref[0]
    tcol = t_ref[0]
    mcol = mask_ref[0]
    tsr4 = ts4_ref[0]
    kc = kc_ref[0]
    vo = vo_ref[0]
    bsc = bsc_ref[0]
    dt_d = dtd_ref[0, 0]

    delta = jnp.abs(tcol - tsr4)
    wmask = jnp.logical_and(delta <= dt_d, mcol != 0.0)

    scale = 1.0 / math.sqrt(HD)
    sca = (jnp.dot(q, kc, preferred_element_type=jnp.float32) + bsc) * scale
    sca = jnp.where(wmask, sca, jnp.float32(-1e9))

    acc = jnp.zeros((TN, D), jnp.float32) + bo_ref[...]
    for h in range(NH):
        s = sca[:, h * KE:(h + 1) * KE]
        m = jnp.max(s, axis=1, keepdims=True)
        p = jnp.exp(s - m)
        p = p * (1.0 / jnp.sum(p, axis=1, keepdims=True))
        acc += jnp.dot(p, vo[h * KE:(h + 1) * KE, :],
                       preferred_element_type=jnp.float32)
    out_ref[0] = acc


def kernel(q_rot, spike, time_norm, var_id, mask, params):
    B, N, Dm = q_rot.shape
    nblk = N // TN

    wst, bs = _build_wt(params['seed'])
    wat, ba = _build_wt(params['aggr'])
    wqt, bq = _build_wt(params['fc_q'])
    wkt, bk = _build_wt(params['fc_k'])
    wvt, bv = _build_wt(params['fc_v'])
    wot, bo = _build_wt(params['fc_o'])
    dt_a = jnp.clip(jnp.exp(params['log_dt_aggr']), 0.001, 1.0).reshape(1, 1)
    dt_d = jnp.clip(jnp.exp(params['log_dt_dist']), 0.001, 1.0).reshape(1, 1)

    var_f = var_id.astype(jnp.float32)
    idxf, tsf, vsf = _sc_select(
        spike.reshape(B * N), mask.reshape(B * N),
        time_norm.reshape(B * N), var_f.reshape(B * N), B, N)

    idx_col = idxf.reshape(B, KE, 1)
    ts_col = tsf.reshape(B, KE, 1)
    vs_col = vsf.reshape(B, KE, 1)

    spike_row = spike.reshape(B, 1, N)
    t_row = time_norm.reshape(B, 1, N)
    var_row = var_f.reshape(B, 1, N)
    mask_row = mask.reshape(B, 1, N)

    col_spec = pl.BlockSpec((1, KE, 1), lambda b, i: (b, 0, 0))
    blk_row_spec = pl.BlockSpec((1, 1, TN), lambda b, i: (b, 0, i))
    w_spec = pl.BlockSpec((D, D), lambda b, i: (0, 0))
    b_spec = pl.BlockSpec((1, D), lambda b, i: (0, 0))
    smem_spec = pl.BlockSpec(memory_space=pltpu.SMEM)

    kc_ev, vo_ev, bsc_ev = pl.pallas_call(
        functools.partial(_events_body, nblk=nblk),
        grid=(B, nblk),
        in_specs=[
            col_spec, col_spec, col_spec,
            blk_row_spec, blk_row_spec, blk_row_spec, blk_row_spec,
            pl.BlockSpec((1, TN, D), lambda b, i: (b, i, 0)),
            smem_spec,
            w_spec, b_spec, w_spec, b_spec, w_spec, b_spec, w_spec, b_spec,
            w_spec, b_spec, w_spec,
        ],
        out_specs=[
            pl.BlockSpec((1, D, NH * KE), lambda b, i: (b, 0, 0)),
            pl.BlockSpec((1, NH * KE, D), lambda b, i: (b, 0, 0)),
            pl.BlockSpec((1, 1, NH * KE), lambda b, i: (b, 0, 0)),
        ],
        out_shape=[
            jax.ShapeDtypeStruct((B, D, NH * KE), jnp.float32),
            jax.ShapeDtypeStruct((B, NH * KE, D), jnp.float32),
            jax.ShapeDtypeStruct((B, 1, NH * KE), jnp.float32),
        ],
        scratch_shapes=[
            pltpu.VMEM((KE, D), jnp.float32),
            pltpu.VMEM((KE, 1), jnp.float32),
            pltpu.VMEM((KE, D), jnp.float32),
        ],
    )(idx_col, ts_col, vs_col,
      spike_row, t_row, var_row, mask_row, q_rot, dt_a,
      wst, bs, wat, ba, wkt, bk, wvt, bv, wqt, bq, wot)

    ts4_row = jnp.concatenate([tsf.reshape(B, 1, KE)] * NH, axis=2)
    t_col = time_norm.reshape(B, N, 1)
    mask_col = mask.reshape(B, N, 1)

    out = pl.pallas_call(
        _attn_body,
        grid=(B, nblk),
        in_specs=[
            pl.BlockSpec((1, TN, D), lambda b, i: (b, i, 0)),
            pl.BlockSpec((1, TN, 1), lambda b, i: (b, i, 0)),
            pl.BlockSpec((1, TN, 1), lambda b, i: (b, i, 0)),
            pl.BlockSpec((1, 1, NH * KE), lambda b, i: (b, 0, 0)),
            pl.BlockSpec((1, D, NH * KE), lambda b, i: (b, 0, 0)),
            pl.BlockSpec((1, NH * KE, D), lambda b, i: (b, 0, 0)),
            pl.BlockSpec((1, 1, NH * KE), lambda b, i: (b, 0, 0)),
            smem_spec,
            b_spec,
        ],
        out_specs=pl.BlockSpec((1, TN, D), lambda b, i: (b, i, 0)),
        out_shape=jax.ShapeDtypeStruct((B, N, Dm), jnp.float32),
    )(q_rot, t_col, mask_col, ts4_row, kc_ev, vo_ev, bsc_ev, dt_d, bo)

    return out


# speedup vs baseline: 35.5591x; 1.0002x over previous
import functools
import math

import jax
import jax.numpy as jnp
from jax import lax
from jax.experimental import pallas as pl
from jax.experimental.pallas import tpu as pltpu
from jax.experimental.pallas import tpu_sc as plsc

D = 256
KE = 128
NH = 4
HD = D // NH
TN = 1024
NPART = 4


def _build_wt(p):
    r, i, j, k = p['r'], p['i'], p['j'], p['k']
    W = jnp.concatenate([
        jnp.concatenate([r, -i, -j, -k], 1),
        jnp.concatenate([i, r, -k, j], 1),
        jnp.concatenate([j, k, r, -i], 1),
        jnp.concatenate([k, -j, i, r], 1),
    ], 0)
    return W.T, p['b'].reshape(1, -1)


def _bs_tau(kref, nv):
    def outer(_, carry):
        lo, hi = carry
        mid = (lo & hi) + ((lo ^ hi) >> 1)

        def inner(i, acc):
            for u in range(4):
                kv = kref[pl.ds(i * 64 + u * 16, 16)]
                acc = acc + jnp.where(kv >= mid, 1, 0).astype(jnp.int32)
            return acc

        accv = lax.fori_loop(0, nv // 4, inner, jnp.zeros((16,), jnp.int32))
        ge = jnp.sum(accv) >= KE
        return (jnp.where(ge, mid, lo), jnp.where(ge, hi, mid))

    lo, _ = lax.fori_loop(0, 32, outer,
                          (jnp.int32(-2**31), jnp.int32(2**31 - 1)))
    return lo


def _count_gt(kref, nv, tau):
    def inner(i, acc):
        for u in range(4):
            kv = kref[pl.ds(i * 64 + u * 16, 16)]
            acc = acc + jnp.where(kv > tau, 1, 0).astype(jnp.int32)
        return acc

    return jnp.sum(lax.fori_loop(0, nv // 4, inner,
                                 jnp.zeros((16,), jnp.int32)))


def _compact(kref, nv, tau, r_need, write_fn):
    def body(i, carry):
        nsel, eqseen = carry
        kv = kref[pl.ds(i * 16, 16)]
        gt = kv > tau
        eq = kv == tau
        eqi = jnp.where(eq, 1, 0).astype(jnp.int32)
        eqpos = eqseen + plsc.cumsum(eqi) - eqi
        sel = gt | (eq & (eqpos < r_need))
        seli = jnp.where(sel, 1, 0).astype(jnp.int32)
        slot = nsel + plsc.cumsum(seli) - seli
        write_fn(i, slot, sel)
        return (nsel + jnp.sum(seli), eqseen + jnp.sum(eqi))

    lax.fori_loop(0, nv, body, (jnp.int32(0), jnp.int32(0)))


def _sc_select(spike_f, mask_f, t_f, var_f, B, N):
    nloc = N // NPART
    ncand = NPART * KE
    mesh = plsc.VectorSubcoreMesh(core_axis_name="c", subcore_axis_name="s")

    @functools.partial(
        pl.kernel,
        out_type=[jax.ShapeDtypeStruct((B * KE,), jnp.float32)] * 3,
        mesh=mesh,
        compiler_params=pltpu.CompilerParams(needs_layout_passes=False),
        scratch_types=[
            pltpu.VMEM((nloc,), jnp.float32),
            pltpu.VMEM((nloc,), jnp.float32),
            pltpu.VMEM((nloc,), jnp.int32),
            pltpu.VMEM((KE,), jnp.int32),
            pltpu.VMEM((KE,), jnp.int32),
            pltpu.VMEM_SHARED((B * NPART * KE,), jnp.int32),
            pltpu.VMEM_SHARED((B * NPART * KE,), jnp.int32),
            pltpu.VMEM((ncand,), jnp.int32),
            pltpu.VMEM((ncand,), jnp.int32),
            pltpu.VMEM((N,), jnp.float32),
            pltpu.VMEM((N,), jnp.float32),
            pltpu.VMEM((KE,), jnp.float32),
            pltpu.VMEM((KE,), jnp.float32),
            pltpu.VMEM((KE,), jnp.float32),
        ],
    )
    def sel(spike_h, mask_h, t_h, var_h, oidx_h, ots_h, ovs_h,
            sp_v, mk_v, key_v, ck_v, ci_v, shk, shi, mgk_v, mgi_v,
            tv, vv, oi_v, ot_v, ov_v):
        cid = lax.axis_index("c")
        sid = lax.axis_index("s")

        @pl.when(cid == 0)
        def _core0():
            row = sid // NPART
            part = sid % NPART
            base = row * N + part * nloc
            pltpu.sync_copy(spike_h.at[pl.ds(base, nloc)], sp_v)
            pltpu.sync_copy(mask_h.at[pl.ds(base, nloc)], mk_v)

            def keys_body(i, _):
                for u in range(4):
                    o = i * 64 + u * 16
                    s = sp_v[pl.ds(o, 16)] * mk_v[pl.ds(o, 16)]
                    b0 = lax.bitcast_convert_type(s, jnp.int32)
                    key_v[pl.ds(o, 16)] = b0 ^ (
                        lax.shift_right_arithmetic(b0, 31)
                        & jnp.int32(0x7FFFFFFF))
                return 0

            lax.fori_loop(0, nloc // 64, keys_body, 0)

            tau1 = _bs_tau(key_v, nloc // 16)
            rn1 = KE - _count_gt(key_v, nloc // 16, tau1)

            def wr1(i, slot, selm):
                kv = key_v[pl.ds(i * 16, 16)]
                gidx = (part * nloc + i * 16
                        + lax.broadcasted_iota(jnp.int32, (16,), 0))
                plsc.store_scatter(ck_v, [slot], kv, mask=selm)
                plsc.store_scatter(ci_v, [slot], gidx, mask=selm)

            _compact(key_v, nloc // 16, tau1, rn1, wr1)

            pltpu.sync_copy(ck_v, shk.at[pl.ds(sid * KE, KE)])
            pltpu.sync_copy(ci_v, shi.at[pl.ds(sid * KE, KE)])
            plsc.subcore_barrier()

            @pl.when(sid < B)
            def _merge():
                pltpu.sync_copy(shk.at[pl.ds(sid * ncand, ncand)], mgk_v)
                pltpu.sync_copy(shi.at[pl.ds(sid * ncand, ncand)], mgi_v)
                pltpu.sync_copy(t_h.at[pl.ds(sid * N, N)], tv)
                pltpu.sync_copy(var_h.at[pl.ds(sid * N, N)], vv)

                tau2 = _bs_tau(mgk_v, ncand // 16)
                rn2 = KE - _count_gt(mgk_v, ncand // 16, tau2)

                def wr2(i, slot, selm):
                    iv = mgi_v[pl.ds(i * 16, 16)]
                    tg = plsc.load_gather(tv, [iv])
                    vg = plsc.load_gather(vv, [iv])
                    plsc.store_scatter(oi_v, [slot],
                                       iv.astype(jnp.float32), mask=selm)
                    plsc.store_scatter(ot_v, [slot], tg, mask=selm)
                    plsc.store_scatter(ov_v, [slot], vg, mask=selm)

                _compact(mgk_v, ncand // 16, tau2, rn2, wr2)

                pltpu.sync_copy(oi_v, oidx_h.at[pl.ds(sid * KE, KE)])
                pltpu.sync_copy(ot_v, ots_h.at[pl.ds(sid * KE, KE)])
                pltpu.sync_copy(ov_v, ovs_h.at[pl.ds(sid * KE, KE)])

    return sel(spike_f, mask_f, t_f, var_f)


def _events_body(idx_ref, ts_ref, vs_ref, sb_ref, tb_ref, vb_ref, mb_ref,
                 q_ref, dta_ref, wst_ref, bs_ref, wat_ref, ba_ref,
                 wkt_ref, bk_ref, wvt_ref, bv_ref,
                 wqt_ref, bq_ref, wot_ref,
                 kc_out, vo_out, bsc_out,
                 num_scr, den_scr, qseed_scr, *, nblk):
    i = pl.program_id(1)
    q = q_ref[0]
    dt_a = dta_ref[0, 0]

    @pl.when(i == 0)
    def _init():
        num_scr[...] = jnp.zeros_like(num_scr)
        den_scr[...] = jnp.zeros_like(den_scr)
        qseed_scr[...] = jnp.zeros_like(qseed_scr)

    idxc = idx_ref[0]
    tsc = ts_ref[0]
    vsc = vs_ref[0]
    tb = tb_ref[0]
    vb = vb_ref[0]
    sb = sb_ref[0]
    mb = mb_ref[0]

    pos = (jax.lax.broadcasted_iota(jnp.int32, (KE, TN), 1)
           + i * TN).astype(jnp.float32)
    s_blk = (idxc == pos).astype(jnp.float32)
    qseed_scr[...] += jax.lax.dot_general(
        s_blk, q, (((1,), (0,)), ((), ())), preferred_element_type=jnp.float32)

    delta = jnp.abs(tb - tsc)
    win = (delta <= dt_a).astype(jnp.float32)
    decay = jnp.exp(-2.0 * delta / jnp.maximum(dt_a, 0.001))
    vaff = 0.5 + 0.5 * (vb == vsc).astype(jnp.float32)
    incid = win * decay * vaff * sb * mb
    num_scr[...] += jax.lax.dot_general(
        incid, q, (((1,), (0,)), ((), ())), preferred_element_type=jnp.float32)
    den_scr[...] += jnp.sum(incid, axis=1, keepdims=True)

    @pl.when(i == nblk - 1)
    def _fin():
        hs = jnp.dot(qseed_scr[...], wst_ref[...],
                     preferred_element_type=jnp.float32) + bs_ref[...]
        agg = num_scr[...] / jnp.maximum(den_scr[...], 1e-6)
        he = hs + jnp.dot(agg, wat_ref[...],
                          preferred_element_type=jnp.float32) + ba_ref[...]
        kk = jnp.dot(he, wkt_ref[...],
                     preferred_element_type=jnp.float32) + bk_ref[...]
        vv = jnp.dot(he, wvt_ref[...],
                     preferred_element_type=jnp.float32) + bv_ref[...]
        kcs, vos, bscs = [], [], []
        for h in range(NH):
            kh = kk[:, h * HD:(h + 1) * HD]
            vh = vv[:, h * HD:(h + 1) * HD]
            kcs.append(jax.lax.dot_general(
                wqt_ref[:, pl.ds(h * HD, HD)], kh, (((1,), (1,)), ((), ())),
                preferred_element_type=jnp.float32))
            bscs.append(jax.lax.dot_general(
                bq_ref[:, pl.ds(h * HD, HD)], kh, (((1,), (1,)), ((), ())),
                preferred_element_type=jnp.float32))
            vos.append(jnp.dot(vh, wot_ref[pl.ds(h * HD, HD), :],
                               preferred_element_type=jnp.float32))
        kc_out[0] = jnp.concatenate(kcs, 1)
        vo_out[0] = jnp.concatenate(vos, 0)
        bsc_out[0] = jnp.concatenate(bscs, 1)


def _attn_body(q_ref, t_ref, mask_ref, ts4_ref, kc_ref, vo_ref, bsc_ref,
               dtd_ref, bo_ref, out_ref):
    q = q_ref[0]
    tcol = t_ref[0]
    mcol = mask_ref[0]
    tsr4 = ts4_ref[0]
    kc = kc_ref[0]
    vo = vo_ref[0]
    bsc = bsc_ref[0]
    dt_d = dtd_ref[0, 0]

    delta = jnp.abs(tcol - tsr4)
    wmask = jnp.logical_and(delta <= dt_d, mcol != 0.0)

    scale = 1.0 / math.sqrt(HD)
    sca = (jnp.dot(q, kc, preferred_element_type=jnp.float32) + bsc) * scale
    sca = jnp.where(wmask, sca, jnp.float32(-1e9))

    acc = jnp.zeros((TN, D), jnp.float32) + bo_ref[...]
    for h in range(NH):
        s = sca[:, h * KE:(h + 1) * KE]
        m = jnp.max(s, axis=1, keepdims=True)
        p = jnp.exp(s - m)
        p = p * (1.0 / jnp.sum(p, axis=1, keepdims=True))
        acc += jnp.dot(p, vo[h * KE:(h + 1) * KE, :],
                       preferred_element_type=jnp.float32)
    out_ref[0] = acc


def kernel(q_rot, spike, time_norm, var_id, mask, params):
    B, N, Dm = q_rot.shape
    nblk = N // TN

    wst, bs = _build_wt(params['seed'])
    wat, ba = _build_wt(params['aggr'])
    wqt, bq = _build_wt(params['fc_q'])
    wkt, bk = _build_wt(params['fc_k'])
    wvt, bv = _build_wt(params['fc_v'])
    wot, bo = _build_wt(params['fc_o'])
    dt_a = jnp.clip(jnp.exp(params['log_dt_aggr']), 0.001, 1.0).reshape(1, 1)
    dt_d = jnp.clip(jnp.exp(params['log_dt_dist']), 0.001, 1.0).reshape(1, 1)

    var_f = var_id.astype(jnp.float32)
    idxf, tsf, vsf = _sc_select(
        spike.reshape(B * N), mask.reshape(B * N),
        time_norm.reshape(B * N), var_f.reshape(B * N), B, N)

    idx_col = idxf.reshape(B, KE, 1)
    ts_col = tsf.reshape(B, KE, 1)
    vs_col = vsf.reshape(B, KE, 1)

    spike_row = spike.reshape(B, 1, N)
    t_row = time_norm.reshape(B, 1, N)
    var_row = var_f.reshape(B, 1, N)
    mask_row = mask.reshape(B, 1, N)

    col_spec = pl.BlockSpec((1, KE, 1), lambda b, i: (b, 0, 0))
    blk_row_spec = pl.BlockSpec((1, 1, TN), lambda b, i: (b, 0, i))
    w_spec = pl.BlockSpec((D, D), lambda b, i: (0, 0))
    b_spec = pl.BlockSpec((1, D), lambda b, i: (0, 0))
    smem_spec = pl.BlockSpec(memory_space=pltpu.SMEM)

    kc_ev, vo_ev, bsc_ev = pl.pallas_call(
        functools.partial(_events_body, nblk=nblk),
        grid=(B, nblk),
        in_specs=[
            col_spec, col_spec, col_spec,
            blk_row_spec, blk_row_spec, blk_row_spec, blk_row_spec,
            pl.BlockSpec((1, TN, D), lambda b, i: (b, i, 0)),
            smem_spec,
            w_spec, b_spec, w_spec, b_spec, w_spec, b_spec, w_spec, b_spec,
            w_spec, b_spec, w_spec,
        ],
        out_specs=[
            pl.BlockSpec((1, D, NH * KE), lambda b, i: (b, 0, 0)),
            pl.BlockSpec((1, NH * KE, D), lambda b, i: (b, 0, 0)),
            pl.BlockSpec((1, 1, NH * KE), lambda b, i: (b, 0, 0)),
        ],
        out_shape=[
            jax.ShapeDtypeStruct((B, D, NH * KE), jnp.float32),
            jax.ShapeDtypeStruct((B, NH * KE, D), jnp.float32),
            jax.ShapeDtypeStruct((B, 1, NH * KE), jnp.float32),
        ],
        scratch_shapes=[
            pltpu.VMEM((KE, D), jnp.float32),
            pltpu.VMEM((KE, 1), jnp.float32),
            pltpu.VMEM((KE, D), jnp.float32),
        ],
    )(idx_col, ts_col, vs_col,
      spike_row, t_row, var_row, mask_row, q_rot, dt_a,
      wst, bs, wat, ba, wkt, bk, wvt, bv, wqt, bq, wot)

    ts4_row = jnp.concatenate([tsf.reshape(B, 1, KE)] * NH, axis=2)
    t_col = time_norm.reshape(B, N, 1)
    mask_col = mask.reshape(B, N, 1)

    out = pl.pallas_call(
        _attn_body,
        grid=(B, nblk),
        in_specs=[
            pl.BlockSpec((1, TN, D), lambda b, i: (b, i, 0)),
            pl.BlockSpec((1, TN, 1), lambda b, i: (b, i, 0)),
            pl.BlockSpec((1, TN, 1), lambda b, i: (b, i, 0)),
            pl.BlockSpec((1, 1, NH * KE), lambda b, i: (b, 0, 0)),
            pl.BlockSpec((1, D, NH * KE), lambda b, i: (b, 0, 0)),
            pl.BlockSpec((1, NH * KE, D), lambda b, i: (b, 0, 0)),
            pl.BlockSpec((1, 1, NH * KE), lambda b, i: (b, 0, 0)),
            smem_spec,
            b_spec,
        ],
        out_specs=pl.BlockSpec((1, TN, D), lambda b, i: (b, i, 0)),
        out_shape=jax.ShapeDtypeStruct((B, N, Dm), jnp.float32),
    )(q_rot, t_col, mask_col, ts4_row, kc_ev, vo_ev, bsc_ev, dt_d, bo)

    return out


# speedup vs baseline: 35.7908x; 1.0065x over previous
import functools
import math

import jax
import jax.numpy as jnp
from jax import lax
from jax.experimental import pallas as pl
from jax.experimental.pallas import tpu as pltpu
from jax.experimental.pallas import tpu_sc as plsc

D = 256
KE = 128
NH = 4
HD = D // NH
TN = 1024
NPART = 4


def _build_wt(p):
    r, i, j, k = p['r'], p['i'], p['j'], p['k']
    W = jnp.concatenate([
        jnp.concatenate([r, -i, -j, -k], 1),
        jnp.concatenate([i, r, -k, j], 1),
        jnp.concatenate([j, k, r, -i], 1),
        jnp.concatenate([k, -j, i, r], 1),
    ], 0)
    return W.T, p['b'].reshape(1, -1)


def _bs_tau(kref, nv):
    def outer(_, carry):
        lo, hi = carry
        mid = (lo & hi) + ((lo ^ hi) >> 1)

        def inner(i, acc):
            for u in range(4):
                kv = kref[pl.ds(i * 64 + u * 16, 16)]
                acc = acc + jnp.where(kv >= mid, 1, 0).astype(jnp.int32)
            return acc

        accv = lax.fori_loop(0, nv // 4, inner, jnp.zeros((16,), jnp.int32))
        ge = jnp.sum(accv) >= KE
        return (jnp.where(ge, mid, lo), jnp.where(ge, hi, mid))

    lo, _ = lax.fori_loop(0, 32, outer,
                          (jnp.int32(-2**31), jnp.int32(2**31 - 1)))
    return lo


def _count_gt(kref, nv, tau):
    def inner(i, acc):
        for u in range(4):
            kv = kref[pl.ds(i * 64 + u * 16, 16)]
            acc = acc + jnp.where(kv > tau, 1, 0).astype(jnp.int32)
        return acc

    return jnp.sum(lax.fori_loop(0, nv // 4, inner,
                                 jnp.zeros((16,), jnp.int32)))


def _compact(kref, nv, tau, r_need, write_fn):
    def body(i, carry):
        nsel, eqseen = carry
        kv = kref[pl.ds(i * 16, 16)]
        gt = kv > tau
        eq = kv == tau
        eqi = jnp.where(eq, 1, 0).astype(jnp.int32)
        eqpos = eqseen + plsc.cumsum(eqi) - eqi
        sel = gt | (eq & (eqpos < r_need))
        seli = jnp.where(sel, 1, 0).astype(jnp.int32)
        slot = nsel + plsc.cumsum(seli) - seli
        write_fn(i, slot, sel)
        return (nsel + jnp.sum(seli), eqseen + jnp.sum(eqi))

    lax.fori_loop(0, nv, body, (jnp.int32(0), jnp.int32(0)))


def _sc_select(spike_f, mask_f, t_f, var_f, B, N):
    nloc = N // NPART
    ncand = NPART * KE
    mesh = plsc.VectorSubcoreMesh(core_axis_name="c", subcore_axis_name="s")

    @functools.partial(
        pl.kernel,
        out_type=[jax.ShapeDtypeStruct((B * KE,), jnp.float32)] * 3,
        mesh=mesh,
        compiler_params=pltpu.CompilerParams(needs_layout_passes=False),
        scratch_types=[
            pltpu.VMEM((nloc,), jnp.float32),
            pltpu.VMEM((nloc,), jnp.float32),
            pltpu.VMEM((nloc,), jnp.int32),
            pltpu.VMEM((KE,), jnp.int32),
            pltpu.VMEM((KE,), jnp.int32),
            pltpu.VMEM_SHARED((B * NPART * KE,), jnp.int32),
            pltpu.VMEM_SHARED((B * NPART * KE,), jnp.int32),
            pltpu.VMEM((ncand,), jnp.int32),
            pltpu.VMEM((ncand,), jnp.int32),
            pltpu.VMEM((N,), jnp.float32),
            pltpu.VMEM((N,), jnp.float32),
            pltpu.VMEM((KE,), jnp.float32),
            pltpu.VMEM((KE,), jnp.float32),
            pltpu.VMEM((KE,), jnp.float32),
        ],
    )
    def sel(spike_h, mask_h, t_h, var_h, oidx_h, ots_h, ovs_h,
            sp_v, mk_v, key_v, ck_v, ci_v, shk, shi, mgk_v, mgi_v,
            tv, vv, oi_v, ot_v, ov_v):
        cid = lax.axis_index("c")
        sid = lax.axis_index("s")

        @pl.when(cid == 0)
        def _core0():
            row = sid // NPART
            part = sid % NPART
            base = row * N + part * nloc
            pltpu.sync_copy(spike_h.at[pl.ds(base, nloc)], sp_v)
            pltpu.sync_copy(mask_h.at[pl.ds(base, nloc)], mk_v)

            def keys_body(i, _):
                for u in range(4):
                    o = i * 64 + u * 16
                    s = sp_v[pl.ds(o, 16)] * mk_v[pl.ds(o, 16)]
                    b0 = lax.bitcast_convert_type(s, jnp.int32)
                    key_v[pl.ds(o, 16)] = b0 ^ (
                        lax.shift_right_arithmetic(b0, 31)
                        & jnp.int32(0x7FFFFFFF))
                return 0

            lax.fori_loop(0, nloc // 64, keys_body, 0)

            tau1 = _bs_tau(key_v, nloc // 16)
            rn1 = KE - _count_gt(key_v, nloc // 16, tau1)

            def wr1(i, slot, selm):
                kv = key_v[pl.ds(i * 16, 16)]
                gidx = (part * nloc + i * 16
                        + lax.broadcasted_iota(jnp.int32, (16,), 0))
                plsc.store_scatter(ck_v, [slot], kv, mask=selm)
                plsc.store_scatter(ci_v, [slot], gidx, mask=selm)

            _compact(key_v, nloc // 16, tau1, rn1, wr1)

            pltpu.sync_copy(ck_v, shk.at[pl.ds(sid * KE, KE)])
            pltpu.sync_copy(ci_v, shi.at[pl.ds(sid * KE, KE)])
            plsc.subcore_barrier()

            @pl.when(sid < B)
            def _merge():
                pltpu.sync_copy(shk.at[pl.ds(sid * ncand, ncand)], mgk_v)
                pltpu.sync_copy(shi.at[pl.ds(sid * ncand, ncand)], mgi_v)
                pltpu.sync_copy(t_h.at[pl.ds(sid * N, N)], tv)
                pltpu.sync_copy(var_h.at[pl.ds(sid * N, N)], vv)

                tau2 = _bs_tau(mgk_v, ncand // 16)
                rn2 = KE - _count_gt(mgk_v, ncand // 16, tau2)

                def wr2(i, slot, selm):
                    iv = mgi_v[pl.ds(i * 16, 16)]
                    tg = plsc.load_gather(tv, [iv])
                    vg = plsc.load_gather(vv, [iv])
                    plsc.store_scatter(oi_v, [slot],
                                       iv.astype(jnp.float32), mask=selm)
                    plsc.store_scatter(ot_v, [slot], tg, mask=selm)
                    plsc.store_scatter(ov_v, [slot], vg, mask=selm)

                _compact(mgk_v, ncand // 16, tau2, rn2, wr2)

                pltpu.sync_copy(oi_v, oidx_h.at[pl.ds(sid * KE, KE)])
                pltpu.sync_copy(ot_v, ots_h.at[pl.ds(sid * KE, KE)])
                pltpu.sync_copy(ov_v, ovs_h.at[pl.ds(sid * KE, KE)])

    return sel(spike_f, mask_f, t_f, var_f)


def _events_body(idx_ref, ts_ref, vs_ref, sb_ref, tb_ref, vb_ref, mb_ref,
                 q_ref, dta_ref, wst_ref, bs_ref, wat_ref, ba_ref,
                 wkt_ref, bk_ref, wvt_ref, bv_ref,
                 wqt_ref, bq_ref, wot_ref,
                 kc_out, vo_out, bsc_out,
                 num_scr, den_scr, qseed_scr, *, nblk):
    i = pl.program_id(1)
    q = q_ref[0]
    dt_a = dta_ref[0, 0]

    @pl.when(i == 0)
    def _init():
        num_scr[...] = jnp.zeros_like(num_scr)
        den_scr[...] = jnp.zeros_like(den_scr)
        qseed_scr[...] = jnp.zeros_like(qseed_scr)

    idxc = idx_ref[0]
    tsc = ts_ref[0]
    vsc = vs_ref[0]
    tb = tb_ref[0]
    vb = vb_ref[0]
    sb = sb_ref[0]
    mb = mb_ref[0]

    pos = (jax.lax.broadcasted_iota(jnp.int32, (KE, TN), 1)
           + i * TN).astype(jnp.float32)
    s_blk = (idxc == pos).astype(jnp.float32)
    qseed_scr[...] += jax.lax.dot_general(
        s_blk, q, (((1,), (0,)), ((), ())), preferred_element_type=jnp.float32)

    delta = jnp.abs(tb - tsc)
    win = (delta <= dt_a).astype(jnp.float32)
    decay = jnp.exp(-2.0 * delta / jnp.maximum(dt_a, 0.001))
    vaff = 0.5 + 0.5 * (vb == vsc).astype(jnp.float32)
    incid = win * decay * vaff * sb * mb
    num_scr[...] += jax.lax.dot_general(
        incid, q, (((1,), (0,)), ((), ())), preferred_element_type=jnp.float32)
    den_scr[...] += jnp.sum(incid, axis=1, keepdims=True)

    @pl.when(i == nblk - 1)
    def _fin():
        hs = jnp.dot(qseed_scr[...], wst_ref[...],
                     preferred_element_type=jnp.float32) + bs_ref[...]
        agg = num_scr[...] / jnp.maximum(den_scr[...], 1e-6)
        he = hs + jnp.dot(agg, wat_ref[...],
                          preferred_element_type=jnp.float32) + ba_ref[...]
        kk = jnp.dot(he, wkt_ref[...],
                     preferred_element_type=jnp.float32) + bk_ref[...]
        vv = jnp.dot(he, wvt_ref[...],
                     preferred_element_type=jnp.float32) + bv_ref[...]
        kcs, vos, bscs = [], [], []
        for h in range(NH):
            kh = kk[:, h * HD:(h + 1) * HD]
            vh = vv[:, h * HD:(h + 1) * HD]
            kcs.append(jax.lax.dot_general(
                wqt_ref[:, pl.ds(h * HD, HD)], kh, (((1,), (1,)), ((), ())),
                preferred_element_type=jnp.float32))
            bscs.append(jax.lax.dot_general(
                bq_ref[:, pl.ds(h * HD, HD)], kh, (((1,), (1,)), ((), ())),
                preferred_element_type=jnp.float32))
            vos.append(jnp.dot(vh, wot_ref[pl.ds(h * HD, HD), :],
                               preferred_element_type=jnp.float32))
        kc_out[0] = jnp.concatenate(kcs, 1)
        vo_out[0] = jnp.concatenate(vos, 0)
        bsc_out[0] = jnp.concatenate(bscs, 1)


def _attn_body(q_ref, t_ref, mask_ref, ts4_ref, kc_ref, vo_ref, bsc_ref,
               dtd_ref, bo_ref, out_ref):
    q = q_ref[0]
    tcol = t_ref[0]
    mcol = mask_ref[0]
    tsr = ts4_ref[0]
    tsr4 = jnp.concatenate([tsr] * NH, axis=1)
    kc = kc_ref[0]
    vo = vo_ref[0]
    bsc = bsc_ref[0]
    dt_d = dtd_ref[0, 0]

    delta = jnp.abs(tcol - tsr4)
    wmask = jnp.logical_and(delta <= dt_d, mcol != 0.0)

    scale = 1.0 / math.sqrt(HD)
    sca = (jnp.dot(q, kc, preferred_element_type=jnp.float32) + bsc) * scale
    sca = jnp.where(wmask, sca, jnp.float32(-1e9))

    acc = jnp.zeros((TN, D), jnp.float32) + bo_ref[...]
    for h in range(NH):
        s = sca[:, h * KE:(h + 1) * KE]
        m = jnp.max(s, axis=1, keepdims=True)
        p = jnp.exp(s - m)
        p = p * (1.0 / jnp.sum(p, axis=1, keepdims=True))
        acc += jnp.dot(p, vo[h * KE:(h + 1) * KE, :],
                       preferred_element_type=jnp.float32)
    out_ref[0] = acc


def kernel(q_rot, spike, time_norm, var_id, mask, params):
    B, N, Dm = q_rot.shape
    nblk = N // TN

    wst, bs = _build_wt(params['seed'])
    wat, ba = _build_wt(params['aggr'])
    wqt, bq = _build_wt(params['fc_q'])
    wkt, bk = _build_wt(params['fc_k'])
    wvt, bv = _build_wt(params['fc_v'])
    wot, bo = _build_wt(params['fc_o'])
    dt_a = jnp.clip(jnp.exp(params['log_dt_aggr']), 0.001, 1.0).reshape(1, 1)
    dt_d = jnp.clip(jnp.exp(params['log_dt_dist']), 0.001, 1.0).reshape(1, 1)

    var_f = var_id.astype(jnp.float32)
    idxf, tsf, vsf = _sc_select(
        spike.reshape(B * N), mask.reshape(B * N),
        time_norm.reshape(B * N), var_f.reshape(B * N), B, N)

    idx_col = idxf.reshape(B, KE, 1)
    ts_col = tsf.reshape(B, KE, 1)
    vs_col = vsf.reshape(B, KE, 1)

    spike_row = spike.reshape(B, 1, N)
    t_row = time_norm.reshape(B, 1, N)
    var_row = var_f.reshape(B, 1, N)
    mask_row = mask.reshape(B, 1, N)

    col_spec = pl.BlockSpec((1, KE, 1), lambda b, i: (b, 0, 0))
    blk_row_spec = pl.BlockSpec((1, 1, TN), lambda b, i: (b, 0, i))
    w_spec = pl.BlockSpec((D, D), lambda b, i: (0, 0))
    b_spec = pl.BlockSpec((1, D), lambda b, i: (0, 0))
    smem_spec = pl.BlockSpec(memory_space=pltpu.SMEM)

    kc_ev, vo_ev, bsc_ev = pl.pallas_call(
        functools.partial(_events_body, nblk=nblk),
        grid=(B, nblk),
        in_specs=[
            col_spec, col_spec, col_spec,
            blk_row_spec, blk_row_spec, blk_row_spec, blk_row_spec,
            pl.BlockSpec((1, TN, D), lambda b, i: (b, i, 0)),
            smem_spec,
            w_spec, b_spec, w_spec, b_spec, w_spec, b_spec, w_spec, b_spec,
            w_spec, b_spec, w_spec,
        ],
        out_specs=[
            pl.BlockSpec((1, D, NH * KE), lambda b, i: (b, 0, 0)),
            pl.BlockSpec((1, NH * KE, D), lambda b, i: (b, 0, 0)),
            pl.BlockSpec((1, 1, NH * KE), lambda b, i: (b, 0, 0)),
        ],
        out_shape=[
            jax.ShapeDtypeStruct((B, D, NH * KE), jnp.float32),
            jax.ShapeDtypeStruct((B, NH * KE, D), jnp.float32),
            jax.ShapeDtypeStruct((B, 1, NH * KE), jnp.float32),
        ],
        scratch_shapes=[
            pltpu.VMEM((KE, D), jnp.float32),
            pltpu.VMEM((KE, 1), jnp.float32),
            pltpu.VMEM((KE, D), jnp.float32),
        ],
    )(idx_col, ts_col, vs_col,
      spike_row, t_row, var_row, mask_row, q_rot, dt_a,
      wst, bs, wat, ba, wkt, bk, wvt, bv, wqt, bq, wot)

    ts4_row = tsf.reshape(B, 1, KE)
    t_col = time_norm.reshape(B, N, 1)
    mask_col = mask.reshape(B, N, 1)

    out = pl.pallas_call(
        _attn_body,
        grid=(B, nblk),
        in_specs=[
            pl.BlockSpec((1, TN, D), lambda b, i: (b, i, 0)),
            pl.BlockSpec((1, TN, 1), lambda b, i: (b, i, 0)),
            pl.BlockSpec((1, TN, 1), lambda b, i: (b, i, 0)),
            pl.BlockSpec((1, 1, KE), lambda b, i: (b, 0, 0)),
            pl.BlockSpec((1, D, NH * KE), lambda b, i: (b, 0, 0)),
            pl.BlockSpec((1, NH * KE, D), lambda b, i: (b, 0, 0)),
            pl.BlockSpec((1, 1, NH * KE), lambda b, i: (b, 0, 0)),
            smem_spec,
            b_spec,
        ],
        out_specs=pl.BlockSpec((1, TN, D), lambda b, i: (b, i, 0)),
        out_shape=jax.ShapeDtypeStruct((B, N, Dm), jnp.float32),
    )(q_rot, t_col, mask_col, ts4_row, kc_ev, vo_ev, bsc_ev, dt_d, bo)

    return out
